# baseline (speedup 1.0000x reference)
import functools

import jax
import jax.numpy as jnp
from jax import lax
from jax.experimental import pallas as pl
from jax.experimental.pallas import tpu as pltpu
from jax.experimental.pallas import tpu_sc as plsc

N = 10000
E = 320000
H = 256
HH = 128
G = 64

BN = 400
NB = N // BN
BE = 2000
EB = E // BE
CH = 80
NCHUNK = E // CH
NTILES = 16
CPT = NCHUNK // NTILES
ROWS_PER_TILE = N // NTILES


def _proj_body(x_ref, w_ref, b_ref, out_ref):
    out_ref[...] = (
        jnp.dot(x_ref[...], w_ref[...], preferred_element_type=jnp.float32)
        + b_ref[0]
    )


def _proj_pack_body(x_ref, w_ref, b_ref, out_ref):
    t = (
        jnp.dot(x_ref[...], w_ref[...], preferred_element_type=jnp.float32)
        + b_ref[0]
    )
    tb = t.astype(jnp.bfloat16).reshape(t.shape[0] // 2, 2, HH)
    au = jax.lax.bitcast_convert_type(tb[:, 0, :], jnp.uint16).astype(jnp.uint32)
    bu = jax.lax.bitcast_convert_type(tb[:, 1, :], jnp.uint16).astype(jnp.uint32)
    out_ref[...] = jax.lax.bitcast_convert_type(au | (bu << 16), jnp.int32)


def _proj_split(x, w, b, rows, row_block, pack=False):
    nb = rows // row_block
    din = x.shape[1]
    b2 = b.reshape(2, 1, HH)
    if pack:
        out_block, out_rows, body = row_block // 2, rows, _proj_pack_body
        out_dtype = jnp.int32
    else:
        out_block, out_rows, body = row_block, 2 * rows, _proj_body
        out_dtype = jnp.float32
    return pl.pallas_call(
        body,
        grid=(nb, 2),
        in_specs=[
            pl.BlockSpec((row_block, din), lambda i, j: (i, 0)),
            pl.BlockSpec((din, HH), lambda i, j: (0, j)),
            pl.BlockSpec((1, 1, HH), lambda i, j: (j, 0, 0)),
        ],
        out_specs=pl.BlockSpec((out_block, HH), lambda i, j: (j * nb + i, 0)),
        out_shape=jax.ShapeDtypeStruct((out_rows, HH), out_dtype),
    )(x, w, b2)


_N_OUTER = CPT // 2


def _mp_body(h_hbm, e_hbm, idx_hbm, z_hbm,
             agg_sh, ib0, ib1, be0, be1, bh0, bh1,
             sem_i0, sem_i1, sem_e0, sem_e1, sem_h0, sem_h1,
             sem_s0, sem_s1):
    c = lax.axis_index("c")
    s = lax.axis_index("s")
    rbase = s * ROWS_PER_TILE
    base_chunk = s * CPT
    bufs = ((ib0, be0, bh0, sem_i0, sem_e0, sem_h0, sem_s0),
            (ib1, be1, bh1, sem_i1, sem_e1, sem_h1, sem_s1))

    def issue_idx(j, b):
        ib, _, _, sem_i, _, _, _ = bufs[b]

        @pl.when(j < CPT)
        def _():
            pltpu.async_copy(
                idx_hbm.at[c * NCHUNK + base_chunk + j], ib, sem_i)

    def issue_e(j, b):
        _, buf_e, _, _, sem_e, _, _ = bufs[b]

        @pl.when(j < CPT)
        def _():
            ebase = (base_chunk + j) * (CH // 2)
            pltpu.async_copy(
                e_hbm.at[pl.ds(c * (E // 2) + ebase, CH // 2)], buf_e, sem_e)

    def issue_gather(j, b):
        ib, _, buf_h, sem_i, _, sem_h, sem_s = bufs[b]

        @pl.when(j < CPT)
        def _():
            pltpu.make_async_copy(
                idx_hbm.at[c * NCHUNK + base_chunk + j], ib, sem_i).wait()

            @pl.when(j >= 2)
            def _():
                pltpu.make_async_copy(buf_h, agg_sh.at[ib.at[1]], sem_s).wait()

            pltpu.async_copy(h_hbm.at[ib.at[0]], buf_h, sem_h)

    issue_idx(jnp.int32(0), 0)
    issue_idx(jnp.int32(1), 1)
    issue_e(jnp.int32(0), 0)
    issue_gather(jnp.int32(0), 0)

    pltpu.sync_copy(
        h_hbm.at[pl.ds(c * N + rbase, ROWS_PER_TILE)],
        agg_sh.at[pl.ds(rbase, ROWS_PER_TILE)],
    )
    plsc.subcore_barrier()

    def outer_body(i2, carry):
        for b in range(2):
            j = i2 * 2 + b
            b1 = 1 - b
            issue_e(j + 1, b1)
            issue_gather(j + 1, b1)

            ib, buf_e, buf_h, _, sem_e, sem_h, sem_s = bufs[b]
            ebase = (base_chunk + j) * (CH // 2)
            pltpu.make_async_copy(
                e_hbm.at[pl.ds(c * (E // 2) + ebase, CH // 2)], buf_e,
                sem_e).wait()
            pltpu.make_async_copy(h_hbm.at[ib.at[0]], buf_h, sem_h).wait()

            @plsc.parallel_loop(0, CH // 2, 1, unroll=4)
            def _(i):
                r0 = 2 * i
                r1 = r0 + 1
                for k in range(HH // 16):
                    ew = buf_e[i, pl.ds(k * 16, 16)]
                    ea = plsc.bitcast(ew << 16, jnp.float32)
                    eb = plsc.bitcast(ew & jnp.int32(-65536), jnp.float32)
                    sl = pl.ds(k * 16, 16)
                    buf_h[r0, sl] = jnp.maximum(buf_h[r0, sl] + ea, 0.0)
                    buf_h[r1, sl] = jnp.maximum(buf_h[r1, sl] + eb, 0.0)

            pltpu.async_copy(buf_h, agg_sh.at[ib.at[1]], sem_s, add=True)
            issue_idx(j + 2, b)

        return carry

    lax.fori_loop(0, _N_OUTER, outer_body, 0)

    for b in range(2):
        ib, _, buf_h, _, _, _, sem_s = bufs[b]
        pltpu.make_async_copy(buf_h, agg_sh.at[ib.at[1]], sem_s).wait()

    plsc.subcore_barrier()
    pltpu.sync_copy(
        agg_sh.at[pl.ds(rbase, ROWS_PER_TILE)],
        z_hbm.at[pl.ds(c * N + rbase, ROWS_PER_TILE)],
    )


@functools.cache
def _get_mp_kernel():
    return pl.kernel(
        _mp_body,
        out_type=jax.ShapeDtypeStruct((2 * N, HH), jnp.float32),
        mesh=plsc.VectorSubcoreMesh(core_axis_name="c", subcore_axis_name="s"),
        compiler_params=pltpu.CompilerParams(
            use_tc_tiling_on_sc=False, needs_layout_passes=False),
        scratch_types=[
            pltpu.VMEM_SHARED((N, HH), jnp.float32),
            pltpu.VMEM((2, CH), jnp.int32),
            pltpu.VMEM((2, CH), jnp.int32),
            pltpu.VMEM((CH // 2, HH), jnp.int32),
            pltpu.VMEM((CH // 2, HH), jnp.int32),
            pltpu.VMEM((CH, HH), jnp.float32),
            pltpu.VMEM((CH, HH), jnp.float32),
            pltpu.SemaphoreType.DMA,
            pltpu.SemaphoreType.DMA,
            pltpu.SemaphoreType.DMA,
            pltpu.SemaphoreType.DMA,
            pltpu.SemaphoreType.DMA,
            pltpu.SemaphoreType.DMA,
            pltpu.SemaphoreType.DMA,
            pltpu.SemaphoreType.DMA,
        ],
    )


def _mlp_body(z0_ref, z1_ref, w1_ref, b1_ref, w2_ref, b2_ref, out_ref):
    w1 = w1_ref[...]
    t = jnp.dot(z0_ref[...], w1[:HH], preferred_element_type=jnp.float32)
    t += jnp.dot(z1_ref[...], w1[HH:], preferred_element_type=jnp.float32)
    t = jnp.maximum(t + b1_ref[...], 0.0)
    o = jnp.dot(t, w2_ref[...], preferred_element_type=jnp.float32) + b2_ref[...]
    o = jnp.maximum(o, 0.0)
    out_ref[0] = o[:, :HH]
    out_ref[1] = o[:, HH:]


def _gine_mlp(z, w1, b1, w2, b2):
    out = pl.pallas_call(
        _mlp_body,
        grid=(NB,),
        in_specs=[
            pl.BlockSpec((BN, HH), lambda i: (i, 0)),
            pl.BlockSpec((BN, HH), lambda i: (NB + i, 0)),
            pl.BlockSpec((H, 2 * H), lambda i: (0, 0)),
            pl.BlockSpec((1, 2 * H), lambda i: (0, 0)),
            pl.BlockSpec((2 * H, H), lambda i: (0, 0)),
            pl.BlockSpec((1, H), lambda i: (0, 0)),
        ],
        out_specs=pl.BlockSpec((2, BN, HH), lambda i: (0, i, 0)),
        out_shape=jax.ShapeDtypeStruct((2, N, HH), jnp.float32),
    )(z, z, w1, b1.reshape(1, 2 * H), w2, b2.reshape(1, H))
    return out.reshape(2 * N, HH)


def _pool_head_body(batch_ref, h0_ref, h1_ref, wl1_ref, bl1_ref,
                    wl2_ref, bl2_ref, out_ref, acc_ref, cnt_ref):
    i = pl.program_id(0)

    @pl.when(i == 0)
    def _():
        acc_ref[...] = jnp.zeros_like(acc_ref)
        cnt_ref[...] = jnp.zeros_like(cnt_ref)

    b = batch_ref[0]
    gids = lax.broadcasted_iota(jnp.int32, (G, BN), 0)
    onehot = (b == gids).astype(jnp.float32)
    acc_ref[0] += jnp.dot(onehot, h0_ref[...], preferred_element_type=jnp.float32)
    acc_ref[1] += jnp.dot(onehot, h1_ref[...], preferred_element_type=jnp.float32)
    cnt_ref[...] += jnp.broadcast_to(
        jnp.sum(onehot, axis=1, keepdims=True), (G, HH)
    )

    @pl.when(i == NB - 1)
    def _():
        cnt = jnp.maximum(cnt_ref[...], 1.0)
        p0 = acc_ref[0] / cnt
        p1 = acc_ref[1] / cnt
        wl1 = wl1_ref[...]
        t = jnp.dot(p0, wl1[:HH], preferred_element_type=jnp.float32)
        t += jnp.dot(p1, wl1[HH:], preferred_element_type=jnp.float32)
        t = jnp.maximum(t + bl1_ref[...], 0.0)
        out_ref[...] = (
            jnp.dot(t, wl2_ref[...], preferred_element_type=jnp.float32)
            + bl2_ref[...]
        )


def _pool_head(h, batch3, wl1, bl1, wl2, bl2):
    hw = wl1.shape[1]
    out_d = wl2.shape[1]
    return pl.pallas_call(
        _pool_head_body,
        grid=(NB,),
        in_specs=[
            pl.BlockSpec((1, 1, BN), lambda i: (i, 0, 0)),
            pl.BlockSpec((BN, HH), lambda i: (i, 0)),
            pl.BlockSpec((BN, HH), lambda i: (NB + i, 0)),
            pl.BlockSpec((H, hw), lambda i: (0, 0)),
            pl.BlockSpec((1, hw), lambda i: (0, 0)),
            pl.BlockSpec((hw, out_d), lambda i: (0, 0)),
            pl.BlockSpec((1, out_d), lambda i: (0, 0)),
        ],
        out_specs=pl.BlockSpec((G, out_d), lambda i: (0, 0)),
        out_shape=jax.ShapeDtypeStruct((G, out_d), jnp.float32),
        scratch_shapes=[
            pltpu.VMEM((2, G, HH), jnp.float32),
            pltpu.VMEM((G, HH), jnp.float32),
        ],
    )(batch3, h, h, wl1, bl1.reshape(1, hw), wl2, bl2.reshape(1, out_d))


def kernel(x, edge_index, edge_attr, batch, time,
           W_in, b_in,
           We0, be0, W1_0, b1_0, W2_0, b2_0,
           We1, be1, W1_1, b1_1, W2_1, b2_1,
           Wl1, bl1, Wl2, bl2):
    del time
    ei = edge_index.reshape(2, NCHUNK, CH).transpose(1, 0, 2)
    offs = jnp.array([[0, 0], [N, 0]], jnp.int32).reshape(2, 1, 2, 1)
    idx = (ei[None] + offs).reshape(2 * NCHUNK, 2, CH)
    batch3 = batch.reshape(NB, 1, BN)

    h = _proj_split(x, W_in, b_in, N, BN)
    e0 = _proj_split(edge_attr, We0, be0, E, BE, pack=True)
    e1 = _proj_split(edge_attr, We1, be1, E, BE, pack=True)

    mp = _get_mp_kernel()
    z0 = mp(h, e0, idx)
    h = _gine_mlp(z0, W1_0, b1_0, W2_0, b2_0)
    z1 = mp(h, e1, idx)
    h = _gine_mlp(z1, W1_1, b1_1, W2_1, b2_1)

    return _pool_head(h, batch3, Wl1, bl1, Wl2, bl2)


# speedup vs baseline: 1.1209x; 1.1209x over previous
import functools

import jax
import jax.numpy as jnp
from jax import lax
from jax.experimental import pallas as pl
from jax.experimental.pallas import tpu as pltpu
from jax.experimental.pallas import tpu_sc as plsc

N = 10000
E = 320000
H = 256
HH = 128
G = 64

BN = 400
NB = N // BN
BE = 2000
EB = E // BE
CH = 80
NCHUNK = E // CH
NTILES = 16
CPT = NCHUNK // NTILES
ROWS_PER_TILE = N // NTILES


def _proj_body(x_ref, w_ref, b_ref, out_ref):
    out_ref[...] = (
        jnp.dot(x_ref[...], w_ref[...], preferred_element_type=jnp.float32)
        + b_ref[0]
    )


def _proj_split(x, w, b, rows, row_block):
    nb = rows // row_block
    din = x.shape[1]
    b2 = b.reshape(2, 1, HH)
    return pl.pallas_call(
        _proj_body,
        grid=(nb, 2),
        in_specs=[
            pl.BlockSpec((row_block, din), lambda i, j: (i, 0)),
            pl.BlockSpec((din, HH), lambda i, j: (0, j)),
            pl.BlockSpec((1, 1, HH), lambda i, j: (j, 0, 0)),
        ],
        out_specs=pl.BlockSpec((row_block, HH), lambda i, j: (j * nb + i, 0)),
        out_shape=jax.ShapeDtypeStruct((2 * rows, HH), jnp.float32),
    )(x, w, b2)


_N_OUTER = CPT // 2


def _mp_body(h_hbm, e_hbm, idx_hbm, z_hbm, hb_hbm,
             agg_sh, ib0, ib1, be0, be1, gb0, gb1,
             sem_i0, sem_i1, sem_e0, sem_e1, sem_h0, sem_h1,
             sem_s0, sem_s1):
    c = lax.axis_index("c")
    s = lax.axis_index("s")
    rbase = s * ROWS_PER_TILE
    base_chunk = s * CPT
    bufs = ((ib0, be0, gb0, sem_i0, sem_e0, sem_h0, sem_s0),
            (ib1, be1, gb1, sem_i1, sem_e1, sem_h1, sem_s1))

    def pack_block(off_rows, nrows):
        pltpu.sync_copy(
            h_hbm.at[pl.ds(c * N + rbase + off_rows, nrows)],
            be0.at[pl.ds(0, nrows)])

        def prow(r, _):
            for k in range(HH // 32):
                a = be0[r, pl.ds(32 * k, 16)]
                bb = be0[r, pl.ds(32 * k + 16, 16)]
                pw = plsc.bitcast(
                    plsc.pack(a, bb, format=plsc.PackFormat.INTERLEAVED),
                    jnp.int32)
                gb0[r, pl.ds(16 * k, 16)] = pw
            return _

        lax.fori_loop(0, nrows, prow, 0)
        pltpu.sync_copy(be0.at[pl.ds(0, nrows)],
                        agg_sh.at[pl.ds(rbase + off_rows, nrows)])
        pltpu.sync_copy(gb0.at[pl.ds(0, nrows)],
                        hb_hbm.at[pl.ds(c * N + rbase + off_rows, nrows)])

    _full, _tail = divmod(ROWS_PER_TILE, CH)
    for blk in range(_full):
        pack_block(blk * CH, CH)
    if _tail:
        pack_block(_full * CH, _tail)

    plsc.subcore_barrier()

    def issue_idx(j, b):
        ib, _, _, sem_i, _, _, _ = bufs[b]

        @pl.when(j < CPT)
        def _():
            pltpu.async_copy(
                idx_hbm.at[c * NCHUNK + base_chunk + j], ib, sem_i)

    def issue_e(j, b):
        _, buf_e, _, _, sem_e, _, _ = bufs[b]

        @pl.when(j < CPT)
        def _():
            ebase = (base_chunk + j) * CH
            pltpu.async_copy(e_hbm.at[pl.ds(c * E + ebase, CH)], buf_e, sem_e)

    def issue_gather(j, b):
        ib, _, gb, sem_i, _, sem_h, _ = bufs[b]

        @pl.when(j < CPT)
        def _():
            pltpu.make_async_copy(
                idx_hbm.at[c * NCHUNK + base_chunk + j], ib, sem_i).wait()
            pltpu.async_copy(hb_hbm.at[ib.at[0]], gb, sem_h)

    issue_idx(jnp.int32(0), 0)
    issue_idx(jnp.int32(1), 1)
    issue_e(jnp.int32(0), 0)
    issue_gather(jnp.int32(0), 0)

    def outer_body(i2, carry):
        for b in range(2):
            j = i2 * 2 + b
            b1 = 1 - b

            @pl.when(j >= 1)
            def _():
                ibp, bep, _, _, _, _, sem_sp = bufs[b1]
                pltpu.make_async_copy(
                    bep, agg_sh.at[ibp.at[1]], sem_sp).wait()

            issue_e(j + 1, b1)
            issue_gather(j + 1, b1)

            ib, buf_e, gb, _, sem_e, sem_h, sem_s = bufs[b]
            ebase = (base_chunk + j) * CH
            pltpu.make_async_copy(
                e_hbm.at[pl.ds(c * E + ebase, CH)], buf_e, sem_e).wait()
            pltpu.make_async_copy(hb_hbm.at[ib.at[0]], gb, sem_h).wait()

            @plsc.parallel_loop(0, CH, 1, unroll=4)
            def _(r):
                for k in range(HH // 32):
                    w = gb[r, pl.ds(16 * k, 16)]
                    ha = plsc.bitcast(w << 16, jnp.float32)
                    hb = plsc.bitcast(w & jnp.int32(-65536), jnp.float32)
                    sa = (r, pl.ds(32 * k, 16))
                    sb = (r, pl.ds(32 * k + 16, 16))
                    buf_e[sa] = jnp.maximum(buf_e[sa] + ha, 0.0)
                    buf_e[sb] = jnp.maximum(buf_e[sb] + hb, 0.0)

            pltpu.async_copy(buf_e, agg_sh.at[ib.at[1]], sem_s, add=True)
            issue_idx(j + 2, b)

        return carry

    lax.fori_loop(0, _N_OUTER, outer_body, 0)

    bl = (CPT - 1) % 2
    ibl, bel, _, _, _, _, sem_sl = bufs[bl]
    pltpu.make_async_copy(bel, agg_sh.at[ibl.at[1]], sem_sl).wait()

    plsc.subcore_barrier()
    pltpu.sync_copy(
        agg_sh.at[pl.ds(rbase, ROWS_PER_TILE)],
        z_hbm.at[pl.ds(c * N + rbase, ROWS_PER_TILE)],
    )


@functools.cache
def _get_mp_kernel():
    return pl.kernel(
        _mp_body,
        out_type=(jax.ShapeDtypeStruct((2 * N, HH), jnp.float32),
                  jax.ShapeDtypeStruct((2 * N, HH // 2), jnp.int32)),
        mesh=plsc.VectorSubcoreMesh(core_axis_name="c", subcore_axis_name="s"),
        compiler_params=pltpu.CompilerParams(
            use_tc_tiling_on_sc=False, needs_layout_passes=False),
        scratch_types=[
            pltpu.VMEM_SHARED((N, HH), jnp.float32),
            pltpu.VMEM((2, CH), jnp.int32),
            pltpu.VMEM((2, CH), jnp.int32),
            pltpu.VMEM((CH, HH), jnp.float32),
            pltpu.VMEM((CH, HH), jnp.float32),
            pltpu.VMEM((CH, HH // 2), jnp.int32),
            pltpu.VMEM((CH, HH // 2), jnp.int32),
            pltpu.SemaphoreType.DMA,
            pltpu.SemaphoreType.DMA,
            pltpu.SemaphoreType.DMA,
            pltpu.SemaphoreType.DMA,
            pltpu.SemaphoreType.DMA,
            pltpu.SemaphoreType.DMA,
            pltpu.SemaphoreType.DMA,
            pltpu.SemaphoreType.DMA,
        ],
    )


def _mlp_body(z0_ref, z1_ref, w1_ref, b1_ref, w2_ref, b2_ref, out_ref):
    w1 = w1_ref[...]
    t = jnp.dot(z0_ref[...], w1[:HH], preferred_element_type=jnp.float32)
    t += jnp.dot(z1_ref[...], w1[HH:], preferred_element_type=jnp.float32)
    t = jnp.maximum(t + b1_ref[...], 0.0)
    o = jnp.dot(t, w2_ref[...], preferred_element_type=jnp.float32) + b2_ref[...]
    o = jnp.maximum(o, 0.0)
    out_ref[0] = o[:, :HH]
    out_ref[1] = o[:, HH:]


def _gine_mlp(z, w1, b1, w2, b2):
    out = pl.pallas_call(
        _mlp_body,
        grid=(NB,),
        in_specs=[
            pl.BlockSpec((BN, HH), lambda i: (i, 0)),
            pl.BlockSpec((BN, HH), lambda i: (NB + i, 0)),
            pl.BlockSpec((H, 2 * H), lambda i: (0, 0)),
            pl.BlockSpec((1, 2 * H), lambda i: (0, 0)),
            pl.BlockSpec((2 * H, H), lambda i: (0, 0)),
            pl.BlockSpec((1, H), lambda i: (0, 0)),
        ],
        out_specs=pl.BlockSpec((2, BN, HH), lambda i: (0, i, 0)),
        out_shape=jax.ShapeDtypeStruct((2, N, HH), jnp.float32),
    )(z, z, w1, b1.reshape(1, 2 * H), w2, b2.reshape(1, H))
    return out.reshape(2 * N, HH)


def _pool_head_body(batch_ref, h0_ref, h1_ref, wl1_ref, bl1_ref,
                    wl2_ref, bl2_ref, out_ref, acc_ref, cnt_ref):
    i = pl.program_id(0)

    @pl.when(i == 0)
    def _():
        acc_ref[...] = jnp.zeros_like(acc_ref)
        cnt_ref[...] = jnp.zeros_like(cnt_ref)

    b = batch_ref[0]
    gids = lax.broadcasted_iota(jnp.int32, (G, BN), 0)
    onehot = (b == gids).astype(jnp.float32)
    acc_ref[0] += jnp.dot(onehot, h0_ref[...], preferred_element_type=jnp.float32)
    acc_ref[1] += jnp.dot(onehot, h1_ref[...], preferred_element_type=jnp.float32)
    cnt_ref[...] += jnp.broadcast_to(
        jnp.sum(onehot, axis=1, keepdims=True), (G, HH)
    )

    @pl.when(i == NB - 1)
    def _():
        cnt = jnp.maximum(cnt_ref[...], 1.0)
        p0 = acc_ref[0] / cnt
        p1 = acc_ref[1] / cnt
        wl1 = wl1_ref[...]
        t = jnp.dot(p0, wl1[:HH], preferred_element_type=jnp.float32)
        t += jnp.dot(p1, wl1[HH:], preferred_element_type=jnp.float32)
        t = jnp.maximum(t + bl1_ref[...], 0.0)
        out_ref[...] = (
            jnp.dot(t, wl2_ref[...], preferred_element_type=jnp.float32)
            + bl2_ref[...]
        )


def _pool_head(h, batch3, wl1, bl1, wl2, bl2):
    hw = wl1.shape[1]
    out_d = wl2.shape[1]
    return pl.pallas_call(
        _pool_head_body,
        grid=(NB,),
        in_specs=[
            pl.BlockSpec((1, 1, BN), lambda i: (i, 0, 0)),
            pl.BlockSpec((BN, HH), lambda i: (i, 0)),
            pl.BlockSpec((BN, HH), lambda i: (NB + i, 0)),
            pl.BlockSpec((H, hw), lambda i: (0, 0)),
            pl.BlockSpec((1, hw), lambda i: (0, 0)),
            pl.BlockSpec((hw, out_d), lambda i: (0, 0)),
            pl.BlockSpec((1, out_d), lambda i: (0, 0)),
        ],
        out_specs=pl.BlockSpec((G, out_d), lambda i: (0, 0)),
        out_shape=jax.ShapeDtypeStruct((G, out_d), jnp.float32),
        scratch_shapes=[
            pltpu.VMEM((2, G, HH), jnp.float32),
            pltpu.VMEM((G, HH), jnp.float32),
        ],
    )(batch3, h, h, wl1, bl1.reshape(1, hw), wl2, bl2.reshape(1, out_d))


def kernel(x, edge_index, edge_attr, batch, time,
           W_in, b_in,
           We0, be0, W1_0, b1_0, W2_0, b2_0,
           We1, be1, W1_1, b1_1, W2_1, b2_1,
           Wl1, bl1, Wl2, bl2):
    del time
    ei = edge_index.reshape(2, NCHUNK, CH).transpose(1, 0, 2)
    offs = jnp.array([[0, 0], [N, 0]], jnp.int32).reshape(2, 1, 2, 1)
    idx = (ei[None] + offs).reshape(2 * NCHUNK, 2, CH)
    batch3 = batch.reshape(NB, 1, BN)

    h = _proj_split(x, W_in, b_in, N, BN)
    e0 = _proj_split(edge_attr, We0, be0, E, BE)
    e1 = _proj_split(edge_attr, We1, be1, E, BE)

    mp = _get_mp_kernel()
    z0, _hb0 = mp(h, e0, idx)
    h = _gine_mlp(z0, W1_0, b1_0, W2_0, b2_0)
    z1, _hb1 = mp(h, e1, idx)
    h = _gine_mlp(z1, W1_1, b1_1, W2_1, b2_1)

    return _pool_head(h, batch3, Wl1, bl1, Wl2, bl2)


# speedup vs baseline: 1.1333x; 1.0111x over previous
import functools

import jax
import jax.numpy as jnp
from jax import lax
from jax.experimental import pallas as pl
from jax.experimental.pallas import tpu as pltpu
from jax.experimental.pallas import tpu_sc as plsc

N = 10000
E = 320000
H = 256
HH = 128
G = 64

BN = 400
NB = N // BN
BE = 2000
EB = E // BE
CH = 100
NCHUNK = E // CH
NTILES = 16
CPT = NCHUNK // NTILES
ROWS_PER_TILE = N // NTILES


def _proj_body(x_ref, w_ref, b_ref, out_ref):
    out_ref[...] = (
        jnp.dot(x_ref[...], w_ref[...], preferred_element_type=jnp.float32)
        + b_ref[0]
    )


def _proj_split(x, w, b, rows, row_block):
    nb = rows // row_block
    din = x.shape[1]
    b2 = b.reshape(2, 1, HH)
    return pl.pallas_call(
        _proj_body,
        grid=(nb, 2),
        in_specs=[
            pl.BlockSpec((row_block, din), lambda i, j: (i, 0)),
            pl.BlockSpec((din, HH), lambda i, j: (0, j)),
            pl.BlockSpec((1, 1, HH), lambda i, j: (j, 0, 0)),
        ],
        out_specs=pl.BlockSpec((row_block, HH), lambda i, j: (j * nb + i, 0)),
        out_shape=jax.ShapeDtypeStruct((2 * rows, HH), jnp.float32),
    )(x, w, b2)


_N_OUTER = CPT // 2


def _mp_body(h_hbm, e_hbm, idx_hbm, z_hbm, hb_hbm,
             agg_sh, ib0, ib1, be0, be1, gb0, gb1,
             sem_i0, sem_i1, sem_e0, sem_e1, sem_h0, sem_h1,
             sem_s0, sem_s1):
    c = lax.axis_index("c")
    s = lax.axis_index("s")
    rbase = s * ROWS_PER_TILE
    base_chunk = s * CPT
    bufs = ((ib0, be0, gb0, sem_i0, sem_e0, sem_h0, sem_s0),
            (ib1, be1, gb1, sem_i1, sem_e1, sem_h1, sem_s1))

    def pack_block(off_rows, nrows):
        pltpu.sync_copy(
            h_hbm.at[pl.ds(c * N + rbase + off_rows, nrows)],
            be0.at[pl.ds(0, nrows)])

        def prow(r, _):
            for k in range(HH // 32):
                a = be0[r, pl.ds(32 * k, 16)]
                bb = be0[r, pl.ds(32 * k + 16, 16)]
                pw = plsc.bitcast(
                    plsc.pack(a, bb, format=plsc.PackFormat.INTERLEAVED),
                    jnp.int32)
                gb0[r, pl.ds(16 * k, 16)] = pw
            return _

        lax.fori_loop(0, nrows, prow, 0)
        pltpu.sync_copy(be0.at[pl.ds(0, nrows)],
                        agg_sh.at[pl.ds(rbase + off_rows, nrows)])
        pltpu.sync_copy(gb0.at[pl.ds(0, nrows)],
                        hb_hbm.at[pl.ds(c * N + rbase + off_rows, nrows)])

    _full, _tail = divmod(ROWS_PER_TILE, CH)
    for blk in range(_full):
        pack_block(blk * CH, CH)
    if _tail:
        pack_block(_full * CH, _tail)

    plsc.subcore_barrier()

    def issue_idx(j, b):
        ib, _, _, sem_i, _, _, _ = bufs[b]

        @pl.when(j < CPT)
        def _():
            pltpu.async_copy(
                idx_hbm.at[c * NCHUNK + base_chunk + j], ib, sem_i)

    def issue_e(j, b):
        _, buf_e, _, _, sem_e, _, _ = bufs[b]

        @pl.when(j < CPT)
        def _():
            ebase = (base_chunk + j) * CH
            pltpu.async_copy(e_hbm.at[pl.ds(c * E + ebase, CH)], buf_e, sem_e)

    def issue_gather(j, b):
        ib, _, gb, sem_i, _, sem_h, _ = bufs[b]

        @pl.when(j < CPT)
        def _():
            pltpu.make_async_copy(
                idx_hbm.at[c * NCHUNK + base_chunk + j], ib, sem_i).wait()
            pltpu.async_copy(hb_hbm.at[ib.at[0]], gb, sem_h)

    issue_idx(jnp.int32(0), 0)
    issue_idx(jnp.int32(1), 1)
    issue_e(jnp.int32(0), 0)
    issue_gather(jnp.int32(0), 0)

    def outer_body(i2, carry):
        for b in range(2):
            j = i2 * 2 + b
            b1 = 1 - b

            @pl.when(j >= 1)
            def _():
                ibp, bep, _, _, _, _, sem_sp = bufs[b1]
                pltpu.make_async_copy(
                    bep, agg_sh.at[ibp.at[1]], sem_sp).wait()

            issue_e(j + 1, b1)
            issue_gather(j + 1, b1)

            ib, buf_e, gb, _, sem_e, sem_h, sem_s = bufs[b]
            ebase = (base_chunk + j) * CH
            pltpu.make_async_copy(
                e_hbm.at[pl.ds(c * E + ebase, CH)], buf_e, sem_e).wait()
            pltpu.make_async_copy(hb_hbm.at[ib.at[0]], gb, sem_h).wait()

            @plsc.parallel_loop(0, CH, 1, unroll=4)
            def _(r):
                for k in range(HH // 32):
                    w = gb[r, pl.ds(16 * k, 16)]
                    ha = plsc.bitcast(w << 16, jnp.float32)
                    hb = plsc.bitcast(w & jnp.int32(-65536), jnp.float32)
                    sa = (r, pl.ds(32 * k, 16))
                    sb = (r, pl.ds(32 * k + 16, 16))
                    buf_e[sa] = jnp.maximum(buf_e[sa] + ha, 0.0)
                    buf_e[sb] = jnp.maximum(buf_e[sb] + hb, 0.0)

            pltpu.async_copy(buf_e, agg_sh.at[ib.at[1]], sem_s, add=True)
            issue_idx(j + 2, b)

        return carry

    lax.fori_loop(0, _N_OUTER, outer_body, 0)

    bl = (CPT - 1) % 2
    ibl, bel, _, _, _, _, sem_sl = bufs[bl]
    pltpu.make_async_copy(bel, agg_sh.at[ibl.at[1]], sem_sl).wait()

    plsc.subcore_barrier()
    pltpu.sync_copy(
        agg_sh.at[pl.ds(rbase, ROWS_PER_TILE)],
        z_hbm.at[pl.ds(c * N + rbase, ROWS_PER_TILE)],
    )


@functools.cache
def _get_mp_kernel():
    return pl.kernel(
        _mp_body,
        out_type=(jax.ShapeDtypeStruct((2 * N, HH), jnp.float32),
                  jax.ShapeDtypeStruct((2 * N, HH // 2), jnp.int32)),
        mesh=plsc.VectorSubcoreMesh(core_axis_name="c", subcore_axis_name="s"),
        compiler_params=pltpu.CompilerParams(
            use_tc_tiling_on_sc=False, needs_layout_passes=False),
        scratch_types=[
            pltpu.VMEM_SHARED((N, HH), jnp.float32),
            pltpu.VMEM((2, CH), jnp.int32),
            pltpu.VMEM((2, CH), jnp.int32),
            pltpu.VMEM((CH, HH), jnp.float32),
            pltpu.VMEM((CH, HH), jnp.float32),
            pltpu.VMEM((CH, HH // 2), jnp.int32),
            pltpu.VMEM((CH, HH // 2), jnp.int32),
            pltpu.SemaphoreType.DMA,
            pltpu.SemaphoreType.DMA,
            pltpu.SemaphoreType.DMA,
            pltpu.SemaphoreType.DMA,
            pltpu.SemaphoreType.DMA,
            pltpu.SemaphoreType.DMA,
            pltpu.SemaphoreType.DMA,
            pltpu.SemaphoreType.DMA,
        ],
    )


def _mlp_body(z0_ref, z1_ref, w1_ref, b1_ref, w2_ref, b2_ref, out_ref):
    w1 = w1_ref[...]
    t = jnp.dot(z0_ref[...], w1[:HH], preferred_element_type=jnp.float32)
    t += jnp.dot(z1_ref[...], w1[HH:], preferred_element_type=jnp.float32)
    t = jnp.maximum(t + b1_ref[...], 0.0)
    o = jnp.dot(t, w2_ref[...], preferred_element_type=jnp.float32) + b2_ref[...]
    o = jnp.maximum(o, 0.0)
    out_ref[0] = o[:, :HH]
    out_ref[1] = o[:, HH:]


def _gine_mlp(z, w1, b1, w2, b2):
    out = pl.pallas_call(
        _mlp_body,
        grid=(NB,),
        in_specs=[
            pl.BlockSpec((BN, HH), lambda i: (i, 0)),
            pl.BlockSpec((BN, HH), lambda i: (NB + i, 0)),
            pl.BlockSpec((H, 2 * H), lambda i: (0, 0)),
            pl.BlockSpec((1, 2 * H), lambda i: (0, 0)),
            pl.BlockSpec((2 * H, H), lambda i: (0, 0)),
            pl.BlockSpec((1, H), lambda i: (0, 0)),
        ],
        out_specs=pl.BlockSpec((2, BN, HH), lambda i: (0, i, 0)),
        out_shape=jax.ShapeDtypeStruct((2, N, HH), jnp.float32),
    )(z, z, w1, b1.reshape(1, 2 * H), w2, b2.reshape(1, H))
    return out.reshape(2 * N, HH)


def _pool_head_body(batch_ref, h0_ref, h1_ref, wl1_ref, bl1_ref,
                    wl2_ref, bl2_ref, out_ref, acc_ref, cnt_ref):
    i = pl.program_id(0)

    @pl.when(i == 0)
    def _():
        acc_ref[...] = jnp.zeros_like(acc_ref)
        cnt_ref[...] = jnp.zeros_like(cnt_ref)

    b = batch_ref[0]
    gids = lax.broadcasted_iota(jnp.int32, (G, BN), 0)
    onehot = (b == gids).astype(jnp.float32)
    acc_ref[0] += jnp.dot(onehot, h0_ref[...], preferred_element_type=jnp.float32)
    acc_ref[1] += jnp.dot(onehot, h1_ref[...], preferred_element_type=jnp.float32)
    cnt_ref[...] += jnp.broadcast_to(
        jnp.sum(onehot, axis=1, keepdims=True), (G, HH)
    )

    @pl.when(i == NB - 1)
    def _():
        cnt = jnp.maximum(cnt_ref[...], 1.0)
        p0 = acc_ref[0] / cnt
        p1 = acc_ref[1] / cnt
        wl1 = wl1_ref[...]
        t = jnp.dot(p0, wl1[:HH], preferred_element_type=jnp.float32)
        t += jnp.dot(p1, wl1[HH:], preferred_element_type=jnp.float32)
        t = jnp.maximum(t + bl1_ref[...], 0.0)
        out_ref[...] = (
            jnp.dot(t, wl2_ref[...], preferred_element_type=jnp.float32)
            + bl2_ref[...]
        )


def _pool_head(h, batch3, wl1, bl1, wl2, bl2):
    hw = wl1.shape[1]
    out_d = wl2.shape[1]
    return pl.pallas_call(
        _pool_head_body,
        grid=(NB,),
        in_specs=[
            pl.BlockSpec((1, 1, BN), lambda i: (i, 0, 0)),
            pl.BlockSpec((BN, HH), lambda i: (i, 0)),
            pl.BlockSpec((BN, HH), lambda i: (NB + i, 0)),
            pl.BlockSpec((H, hw), lambda i: (0, 0)),
            pl.BlockSpec((1, hw), lambda i: (0, 0)),
            pl.BlockSpec((hw, out_d), lambda i: (0, 0)),
            pl.BlockSpec((1, out_d), lambda i: (0, 0)),
        ],
        out_specs=pl.BlockSpec((G, out_d), lambda i: (0, 0)),
        out_shape=jax.ShapeDtypeStruct((G, out_d), jnp.float32),
        scratch_shapes=[
            pltpu.VMEM((2, G, HH), jnp.float32),
            pltpu.VMEM((G, HH), jnp.float32),
        ],
    )(batch3, h, h, wl1, bl1.reshape(1, hw), wl2, bl2.reshape(1, out_d))


def kernel(x, edge_index, edge_attr, batch, time,
           W_in, b_in,
           We0, be0, W1_0, b1_0, W2_0, b2_0,
           We1, be1, W1_1, b1_1, W2_1, b2_1,
           Wl1, bl1, Wl2, bl2):
    del time
    ei = edge_index.reshape(2, NCHUNK, CH).transpose(1, 0, 2)
    offs = jnp.array([[0, 0], [N, 0]], jnp.int32).reshape(2, 1, 2, 1)
    idx = (ei[None] + offs).reshape(2 * NCHUNK, 2, CH)
    batch3 = batch.reshape(NB, 1, BN)

    h = _proj_split(x, W_in, b_in, N, BN)
    e0 = _proj_split(edge_attr, We0, be0, E, BE)
    e1 = _proj_split(edge_attr, We1, be1, E, BE)

    mp = _get_mp_kernel()
    z0, _hb0 = mp(h, e0, idx)
    h = _gine_mlp(z0, W1_0, b1_0, W2_0, b2_0)
    z1, _hb1 = mp(h, e1, idx)
    h = _gine_mlp(z1, W1_1, b1_1, W2_1, b2_1)

    return _pool_head(h, batch3, Wl1, bl1, Wl2, bl2)
